# baseline (speedup 1.0000x reference)
import functools
import numpy as np
import jax
import jax.numpy as jnp
from jax import lax
from jax.experimental import pallas as pl
from jax.experimental.pallas import tpu as pltpu
from jax.experimental.pallas import tpu_sc as plsc

_SIZE = 16
_DATA = [(0, 0, 1), (0, 1, 2), (0, 2, 3), (0, 3, 4), (0, 4, 5)]

_ACTIVE = sorted({t for (t, _, _) in _DATA})
_RUNS = []
for _t in _ACTIVE:
    if _RUNS and _RUNS[-1][-1] == _t - 1:
        _RUNS[-1].append(_t)
    else:
        _RUNS.append([_t])
_NSQ = max(1, int(np.ceil(np.log2(_SIZE))))

_BLOCKS = []
for _run in _RUNS:
    for _i, _t0 in enumerate(_run):
        for _t1 in _run[_i:]:
            _BLOCKS.append((_t0, _t1))

_NW = 16
_OUT = _SIZE ** 4
_NSLICE = 16
_PER_S = _OUT // _NSLICE
_BLK = _SIZE * _SIZE

_SLICE_BLOCKS = {j: [] for j in range(_NSLICE)}
for _t0, _t1 in _BLOCKS:
    _SLICE_BLOCKS[((_t0 * _SIZE + _t1) * _BLK) // _PER_S].append((_t0, _t1))
_COMPUTE_SLICES = [j for j in range(_NSLICE) if _SLICE_BLOCKS[j]]
_ZERO_SLICES = [j for j in range(_NSLICE) if not _SLICE_BLOCKS[j]]
_SLICE_OWNER = {}
for _i, _j in enumerate(_COMPUTE_SLICES):
    _SLICE_OWNER[_j] = _i % _NW
_ZSTART = min(len(_COMPUTE_SLICES), _NW - 1)
_NZW = max(1, _NW - _ZSTART)
for _i, _j in enumerate(_ZERO_SLICES):
    _SLICE_OWNER[_j] = _ZSTART + (_i % _NZW)


def _build_single():
    idx = np.array([i * _SIZE * _SIZE + j * _SIZE + k for (i, j, k) in _DATA],
                   dtype=np.int64)
    s = np.zeros((_SIZE ** 3,), np.float32)
    s[idx] = 0.5
    return jnp.asarray(s.reshape(_SIZE, _SIZE, _SIZE))


def _splat(vec, k):
    return vec.at[jnp.full((_SIZE,), k, jnp.int32)].get(
        mode="promise_in_bounds")


def _mm_rows(a_rows, b_rows):
    out = []
    for i in range(_SIZE):
        acc = None
        for k in range(_SIZE):
            term = jnp.minimum(_splat(a_rows[i], k), b_rows[k])
            acc = term if acc is None else jnp.maximum(acc, term)
        out.append(acc)
    return out


def _closure_rows(rows):
    for _ in range(_NSQ):
        sq = _mm_rows(rows, rows)
        rows = [jnp.maximum(r, s) for r, s in zip(rows, sq)]
    return rows


def _sc_body(single_hbm, out_hbm, zbuf, s_vmem):
    wid = lax.axis_index("s")
    zero = jnp.zeros((_SIZE,), jnp.float32)
    for i in range(_PER_S // _SIZE):
        zbuf[pl.ds(i * _SIZE, _SIZE)] = zero
    for j in range(_NSLICE):
        @pl.when(wid == _SLICE_OWNER[j])
        def _emit_slice(j=j):
            for (t0, t1) in _SLICE_BLOCKS[j]:
                pltpu.sync_copy(single_hbm.at[t0], s_vmem)
                rows = [s_vmem[i, :] for i in range(_SIZE)]
                rows = _closure_rows(rows)
                for t in range(t0 + 1, t1 + 1):
                    pltpu.sync_copy(single_hbm.at[t], s_vmem)
                    b_rows = [s_vmem[i, :] for i in range(_SIZE)]
                    rows = _mm_rows(rows, b_rows)
                base = (t0 * _SIZE + t1) * _BLK - j * _PER_S
                for i in range(_SIZE):
                    zbuf[pl.ds(base + i * _SIZE, _SIZE)] = rows[i]
            pltpu.sync_copy(zbuf, out_hbm.at[pl.ds(j * _PER_S, _PER_S)])
            if len([s for s in range(_NSLICE)
                    if _SLICE_OWNER[s] == _SLICE_OWNER[j]]) > 1:
                for (t0, t1) in _SLICE_BLOCKS[j]:
                    base = (t0 * _SIZE + t1) * _BLK - j * _PER_S
                    for i in range(_SIZE):
                        zbuf[pl.ds(base + i * _SIZE, _SIZE)] = zero


def kernel(x, W1, b1, W2, b2):
    del x, W1, b1, W2, b2
    single = _build_single()
    mesh = plsc.VectorSubcoreMesh(core_axis_name="c", subcore_axis_name="s",
                                  num_cores=1)
    k = functools.partial(
        pl.kernel,
        mesh=mesh,
        out_type=jax.ShapeDtypeStruct((_OUT,), jnp.float32),
        scratch_types=[
            pltpu.VMEM((_PER_S,), jnp.float32),
            pltpu.VMEM((_SIZE, _SIZE), jnp.float32),
        ],
    )(_sc_body)
    out = k(single)
    return out.reshape(1, _OUT)


# speedup vs baseline: 1.1329x; 1.1329x over previous
import functools
import numpy as np
import jax
import jax.numpy as jnp
from jax import lax
from jax.experimental import pallas as pl
from jax.experimental.pallas import tpu as pltpu
from jax.experimental.pallas import tpu_sc as plsc

_SIZE = 16
_DATA = [(0, 0, 1), (0, 1, 2), (0, 2, 3), (0, 3, 4), (0, 4, 5)]

_ACTIVE = sorted({t for (t, _, _) in _DATA})
_RUNS = []
for _t in _ACTIVE:
    if _RUNS and _RUNS[-1][-1] == _t - 1:
        _RUNS[-1].append(_t)
    else:
        _RUNS.append([_t])
_NSQ = max(1, int(np.ceil(np.log2(_SIZE))))

_BLOCKS = []
for _run in _RUNS:
    for _i, _t0 in enumerate(_run):
        for _t1 in _run[_i:]:
            _BLOCKS.append((_t0, _t1))

_NW = 16
_OUT = _SIZE ** 4
_NSLICE = 16
_PER_S = _OUT // _NSLICE
_BLK = _SIZE * _SIZE

_SLICE_BLOCKS = {j: [] for j in range(_NSLICE)}
for _t0, _t1 in _BLOCKS:
    _SLICE_BLOCKS[((_t0 * _SIZE + _t1) * _BLK) // _PER_S].append((_t0, _t1))
_COMPUTE_SLICES = [j for j in range(_NSLICE) if _SLICE_BLOCKS[j]]
_ZERO_SLICES = [j for j in range(_NSLICE) if not _SLICE_BLOCKS[j]]
_SLICE_OWNER = {}
for _i, _j in enumerate(_COMPUTE_SLICES):
    _SLICE_OWNER[_j] = _i % _NW
_ZSTART = min(len(_COMPUTE_SLICES), _NW - 1)
_NZW = max(1, _NW - _ZSTART)
for _i, _j in enumerate(_ZERO_SLICES):
    _SLICE_OWNER[_j] = _ZSTART + (_i % _NZW)


def _splat(vec, k):
    return vec.at[jnp.full((_SIZE,), k, jnp.int32)].get(
        mode="promise_in_bounds")


def _mm_rows(a_rows, b_rows):
    out = []
    for i in range(_SIZE):
        acc = None
        for k in range(_SIZE):
            term = jnp.minimum(_splat(a_rows[i], k), b_rows[k])
            acc = term if acc is None else jnp.maximum(acc, term)
        out.append(acc)
    return out


def _closure_rows(rows):
    for _ in range(_NSQ):
        sq = _mm_rows(rows, rows)
        rows = [jnp.maximum(r, s) for r, s in zip(rows, sq)]
    return rows


_PROB = 0.5
_ROW_LANES = {}
for _t, _v0, _v1 in _DATA:
    _ROW_LANES.setdefault((_t, _v0), []).append(_v1)


def _s_rows(t):
    iota = lax.iota(jnp.int32, _SIZE)
    zero = jnp.zeros((_SIZE,), jnp.float32)
    rows = []
    for i in range(_SIZE):
        vec = zero
        for v1 in _ROW_LANES.get((t, i), ()):
            vec = jnp.where(iota == v1, jnp.float32(_PROB), vec)
        rows.append(vec)
    return rows


def _sc_body(out_hbm, zbuf):
    wid = lax.axis_index("s")
    zero = jnp.zeros((_SIZE,), jnp.float32)
    for i in range(_PER_S // _SIZE):
        zbuf[pl.ds(i * _SIZE, _SIZE)] = zero
    for j in range(_NSLICE):
        @pl.when(wid == _SLICE_OWNER[j])
        def _emit_slice(j=j):
            for (t0, t1) in _SLICE_BLOCKS[j]:
                rows = _closure_rows(_s_rows(t0))
                for t in range(t0 + 1, t1 + 1):
                    rows = _mm_rows(rows, _s_rows(t))
                base = (t0 * _SIZE + t1) * _BLK - j * _PER_S
                for i in range(_SIZE):
                    zbuf[pl.ds(base + i * _SIZE, _SIZE)] = rows[i]
            pltpu.sync_copy(zbuf, out_hbm.at[pl.ds(j * _PER_S, _PER_S)])
            if len([s for s in range(_NSLICE)
                    if _SLICE_OWNER[s] == _SLICE_OWNER[j]]) > 1:
                for (t0, t1) in _SLICE_BLOCKS[j]:
                    base = (t0 * _SIZE + t1) * _BLK - j * _PER_S
                    for i in range(_SIZE):
                        zbuf[pl.ds(base + i * _SIZE, _SIZE)] = zero


def kernel(x, W1, b1, W2, b2):
    del x, W1, b1, W2, b2
    mesh = plsc.VectorSubcoreMesh(core_axis_name="c", subcore_axis_name="s",
                                  num_cores=1)
    k = functools.partial(
        pl.kernel,
        mesh=mesh,
        out_type=jax.ShapeDtypeStruct((_OUT,), jnp.float32),
        scratch_types=[
            pltpu.VMEM((_PER_S,), jnp.float32),
        ],
    )(_sc_body)
    out = k()
    return out.reshape(1, _OUT)


# speedup vs baseline: 1.1456x; 1.0112x over previous
import functools
import numpy as np
import jax
import jax.numpy as jnp
from jax import lax
from jax.experimental import pallas as pl
from jax.experimental.pallas import tpu as pltpu
from jax.experimental.pallas import tpu_sc as plsc

_SIZE = 16
_DATA = [(0, 0, 1), (0, 1, 2), (0, 2, 3), (0, 3, 4), (0, 4, 5)]

_ACTIVE = sorted({t for (t, _, _) in _DATA})
_RUNS = []
for _t in _ACTIVE:
    if _RUNS and _RUNS[-1][-1] == _t - 1:
        _RUNS[-1].append(_t)
    else:
        _RUNS.append([_t])
_EDGES = {t: sum(1 for (tt, _, _) in _DATA if tt == t) for t in _ACTIVE}
_NSQ = {t: max(1, int(np.ceil(np.log2(max(2, min(_SIZE - 1, _EDGES[t]))))))
        for t in _ACTIVE}

_BLOCKS = []
for _run in _RUNS:
    for _i, _t0 in enumerate(_run):
        for _t1 in _run[_i:]:
            _BLOCKS.append((_t0, _t1))

_NW = 16
_OUT = _SIZE ** 4
_NSLICE = 16
_PER_S = _OUT // _NSLICE
_BLK = _SIZE * _SIZE

_SLICE_BLOCKS = {j: [] for j in range(_NSLICE)}
for _t0, _t1 in _BLOCKS:
    _SLICE_BLOCKS[((_t0 * _SIZE + _t1) * _BLK) // _PER_S].append((_t0, _t1))
_COMPUTE_SLICES = [j for j in range(_NSLICE) if _SLICE_BLOCKS[j]]
_ZERO_SLICES = [j for j in range(_NSLICE) if not _SLICE_BLOCKS[j]]
_SLICE_OWNER = {}
for _i, _j in enumerate(_COMPUTE_SLICES):
    _SLICE_OWNER[_j] = _i % _NW
_ZSTART = min(len(_COMPUTE_SLICES), _NW - 1)
_NZW = max(1, _NW - _ZSTART)
for _i, _j in enumerate(_ZERO_SLICES):
    _SLICE_OWNER[_j] = _ZSTART + (_i % _NZW)


def _splat(vec, k):
    return vec.at[jnp.full((_SIZE,), k, jnp.int32)].get(
        mode="promise_in_bounds")


def _mm_rows(a_rows, b_rows):
    out = []
    for i in range(_SIZE):
        acc = None
        for k in range(_SIZE):
            term = jnp.minimum(_splat(a_rows[i], k), b_rows[k])
            acc = term if acc is None else jnp.maximum(acc, term)
        out.append(acc)
    return out


def _closure_rows(rows, t):
    for _ in range(_NSQ[t]):
        sq = _mm_rows(rows, rows)
        rows = [jnp.maximum(r, s) for r, s in zip(rows, sq)]
    return rows


_PROB = 0.5
_ROW_LANES = {}
for _t, _v0, _v1 in _DATA:
    _ROW_LANES.setdefault((_t, _v0), []).append(_v1)


def _s_rows(t):
    iota = lax.iota(jnp.int32, _SIZE)
    zero = jnp.zeros((_SIZE,), jnp.float32)
    rows = []
    for i in range(_SIZE):
        vec = zero
        for v1 in _ROW_LANES.get((t, i), ()):
            vec = jnp.where(iota == v1, jnp.float32(_PROB), vec)
        rows.append(vec)
    return rows


def _sc_body(out_hbm, zbuf):
    wid = lax.axis_index("s")
    zero = jnp.zeros((_SIZE,), jnp.float32)
    for i in range(_PER_S // _SIZE):
        zbuf[pl.ds(i * _SIZE, _SIZE)] = zero
    for j in range(_NSLICE):
        @pl.when(wid == _SLICE_OWNER[j])
        def _emit_slice(j=j):
            for (t0, t1) in _SLICE_BLOCKS[j]:
                rows = _closure_rows(_s_rows(t0), t0)
                for t in range(t0 + 1, t1 + 1):
                    rows = _mm_rows(rows, _s_rows(t))
                base = (t0 * _SIZE + t1) * _BLK - j * _PER_S
                for i in range(_SIZE):
                    zbuf[pl.ds(base + i * _SIZE, _SIZE)] = rows[i]
            pltpu.sync_copy(zbuf, out_hbm.at[pl.ds(j * _PER_S, _PER_S)])
            if len([s for s in range(_NSLICE)
                    if _SLICE_OWNER[s] == _SLICE_OWNER[j]]) > 1:
                for (t0, t1) in _SLICE_BLOCKS[j]:
                    base = (t0 * _SIZE + t1) * _BLK - j * _PER_S
                    for i in range(_SIZE):
                        zbuf[pl.ds(base + i * _SIZE, _SIZE)] = zero


def kernel(x, W1, b1, W2, b2):
    del x, W1, b1, W2, b2
    mesh = plsc.VectorSubcoreMesh(core_axis_name="c", subcore_axis_name="s",
                                  num_cores=1)
    k = functools.partial(
        pl.kernel,
        mesh=mesh,
        out_type=jax.ShapeDtypeStruct((_OUT,), jnp.float32),
        scratch_types=[
            pltpu.VMEM((_PER_S,), jnp.float32),
        ],
    )(_sc_body)
    out = k()
    return out.reshape(1, _OUT)
